# baseline (speedup 1.0000x reference)
import math

import jax
import jax.numpy as jnp
from jax import lax
from jax.experimental import pallas as pl
from jax.experimental.pallas import tpu as pltpu
from jax.experimental.pallas import tpu_sc as plsc

RESO = 16
EDIM = 64
NEMB = 512
CCOST = 0.25
RAD = 1.3
SH_DEG = 2
FDIM = (SH_DEG + 1) ** 2 * 3 + 1
HID = 32
NRAYS = 1024

_units = RAD * 2 / (RESO - 1)
STEP = _units / 8.0
N_INTRS = int(math.sqrt(3) * RAD * 2 / STEP) - 1
NP = 208
NT = NRAYS * NP
NCELL = RESO * RESO * RESO

RA = 64
BLK = 1664
NBLK_B = NT // BLK
NW = 32
PT_W = NT // NW
SCH = 1664
NSCH = PT_W // SCH
NGRP = SCH // 16

_CORNERS = ((0, 0, 0), (1, 0, 0), (0, 1, 0), (1, 1, 0),
            (0, 0, 1), (1, 0, 1), (0, 1, 1), (1, 1, 1))


def _a_body(ro_ref, rd_ref, base_ref, wx_ref, wy_ref, wz_ref, m_ref,
            f0_ref, f1_ref, f2_ref, d0_ref, d1_ref, d2_ref, t_ref):
    ro = ro_ref[...]
    rd = rd_ref[...]
    ox, oy, oz = ro[:, 0:1], ro[:, 1:2], ro[:, 2:3]
    rx, ry, rz = rd[:, 0:1], rd[:, 1:2], rd[:, 2:3]
    norm = jnp.sqrt(rx * rx + ry * ry + rz * rz)
    dnx, dny, dnz = rx / norm, ry / norm, rz / norm

    def axis_ts(o, d):
        sd = jnp.where(jnp.abs(d) < 1e-9, 1e-9, d)
        inv = 1.0 / sd
        t1 = (-RAD - o) * inv
        t2 = (RAD - o) * inv
        return jnp.minimum(t1, t2), jnp.maximum(t1, t2)

    mn0, mx0 = axis_ts(ox, dnx)
    mn1, mx1 = axis_ts(oy, dny)
    mn2, mx2 = axis_ts(oz, dnz)
    tmin = jnp.maximum(jnp.maximum(jnp.maximum(mn0, mn1), mn2), 0.0)
    tmax = jnp.minimum(jnp.minimum(mx0, mx1), mx2)

    lane = lax.broadcasted_iota(jnp.int32, (RA, NP), 1)
    offs = lane.astype(jnp.float32) * STEP
    t = tmin + offs
    px = ox + dnx * t
    py = oy + dny * t
    pz = oz + dnz * t
    mask = ((t < tmax) & (tmax > tmin)
            & (jnp.abs(px) <= RAD) & (jnp.abs(py) <= RAD)
            & (jnp.abs(pz) <= RAD) & (lane < N_INTRS))
    zero = jnp.zeros((RA, NP), jnp.float32)
    px = jnp.where(mask, px, zero)
    py = jnp.where(mask, py, zero)
    pz = jnp.where(mask, pz, zero)

    def cell(p):
        coords = (p / RAD + 1.0) * (0.5 * (RESO - 1))
        c0 = jnp.clip(jnp.floor(coords), 0.0, RESO - 2)
        return c0.astype(jnp.int32), coords - c0

    cx, wx = cell(px)
    cy, wy = cell(py)
    cz, wz = cell(pz)
    base_ref[...] = cz * (RESO * RESO) + cy * RESO + cx
    wx_ref[...] = wx
    wy_ref[...] = wy
    wz_ref[...] = wz
    m_ref[...] = mask.astype(jnp.float32)

    def fine(p):
        pcc = p * (RESO / (RAD * 2)) + RESO / 2
        return (pcc - jnp.floor(pcc)) * 2.0 - 1.0

    f0_ref[...] = fine(px)
    f1_ref[...] = fine(py)
    f2_ref[...] = fine(pz)
    d0_ref[...] = jnp.broadcast_to(dnx, (RA, NP))
    d1_ref[...] = jnp.broadcast_to(dny, (RA, NP))
    d2_ref[...] = jnp.broadcast_to(dnz, (RA, NP))
    t_ref[...] = t


def _stage_a(rays_o, rays_d):
    n_grid = NRAYS // RA
    f32 = jnp.float32
    outs = [jax.ShapeDtypeStruct((NRAYS, NP), jnp.int32)] + \
           [jax.ShapeDtypeStruct((NRAYS, NP), f32)] * 11
    spec = pl.BlockSpec((RA, NP), lambda i: (i, 0))
    return pl.pallas_call(
        _a_body,
        grid=(n_grid,),
        in_specs=[pl.BlockSpec((RA, 3), lambda i: (i, 0))] * 2,
        out_specs=[spec] * 12,
        out_shape=outs,
    )(rays_o, rays_d)


def _sc_body(tbl_hbm, idx_hbm, wx_hbm, wy_hbm, wz_hbm, out_hbm,
             tblq, idxv, wxv, wyv, wzv, outv):
    wid = lax.axis_index("s") * 2 + lax.axis_index("c")
    tile_base = wid * PT_W

    for q in range(4):
        pltpu.sync_copy(tbl_hbm.at[q], tblq)
        for s in range(NSCH):
            base = tile_base + s * SCH
            pltpu.sync_copy(idx_hbm.at[pl.ds(base, SCH)], idxv)
            pltpu.sync_copy(wx_hbm.at[pl.ds(base, SCH)], wxv)
            pltpu.sync_copy(wy_hbm.at[pl.ds(base, SCH)], wyv)
            pltpu.sync_copy(wz_hbm.at[pl.ds(base, SCH)], wzv)

            def body(g, carry):
                off = g * 16
                cellv = idxv[pl.ds(off, 16)]
                wx = wxv[pl.ds(off, 16)]
                wy = wyv[pl.ds(off, 16)]
                wz = wzv[pl.ds(off, 16)]
                ux = 1.0 - wx
                uy = 1.0 - wy
                uz = 1.0 - wz
                pxy = (ux * uy, wx * uy, ux * wy, wx * wy)
                wcorn = [pxy[0] * uz, pxy[1] * uz, pxy[2] * uz, pxy[3] * uz,
                         pxy[0] * wz, pxy[1] * wz, pxy[2] * wz, pxy[3] * wz]
                accs = [None] * 16
                for k, (dx, dy, dz) in enumerate(_CORNERS):
                    cidx = cellv + (dz * (RESO * RESO) + dy * RESO + dx)
                    wk = wcorn[k]
                    for c in range(16):
                        val = plsc.load_gather(tblq, [cidx + c * NCELL])
                        accs[c] = val * wk if k == 0 else accs[c] + val * wk
                for c in range(16):
                    outv[c, pl.ds(off, 16)] = accs[c]
                return carry

            lax.fori_loop(0, NGRP, body, 0)
            pltpu.sync_copy(outv,
                            out_hbm.at[pl.ds(q * 16, 16), pl.ds(base, SCH)])


def _sc_gather(tbl, idx, wx, wy, wz):
    f32 = jnp.float32
    mesh = plsc.VectorSubcoreMesh(core_axis_name="c", subcore_axis_name="s",
                                  num_cores=2, num_subcores=16)
    fn = pl.kernel(
        _sc_body,
        out_type=jax.ShapeDtypeStruct((EDIM, NT), f32),
        mesh=mesh,
        compiler_params=pltpu.CompilerParams(needs_layout_passes=False),
        scratch_types=[
            pltpu.VMEM((16 * NCELL,), f32),
            pltpu.VMEM((SCH,), jnp.int32),
            pltpu.VMEM((SCH,), f32),
            pltpu.VMEM((SCH,), f32),
            pltpu.VMEM((SCH,), f32),
            pltpu.VMEM((16, SCH), f32),
        ],
    )
    return fn(tbl, idx, wx, wy, wz)


def _b_body(cT_ref, m_ref, f0_ref, f1_ref, f2_ref, d0_ref, d1_ref, d2_ref,
            emb_ref, W1_ref, b1_ref, W2_ref, b2_ref, W3_ref, b3_ref,
            sig_ref, r0_ref, r1_ref, r2_ref, comm_ref, perp_ref,
            bins_acc, cnt_acc, en_acc, xT_s):
    i = pl.program_id(0)
    f32 = jnp.float32
    cT = cT_ref[...]
    m = m_ref[...]
    emb = emb_ref[...]

    dn = (((1,), (0,)), ((), ()))
    scores = lax.dot_general(emb.astype(jnp.bfloat16), cT.astype(jnp.bfloat16),
                             dn, preferred_element_type=f32)
    e2 = jnp.sum(emb * emb, axis=1)[:, None]
    d2 = e2 - 2.0 * scores
    minv = jnp.min(d2, axis=0, keepdims=True)
    iota = lax.broadcasted_iota(jnp.int32, (NEMB, BLK), 0)
    idx = jnp.min(jnp.where(d2 == minv, iota, NEMB), axis=0, keepdims=True)
    onehot = (iota == idx).astype(f32)

    dc = (((0,), (0,)), ((), ()))
    qT = lax.dot_general(emb, onehot, dc, preferred_element_type=f32)

    diff = qT - cT
    en_part = jnp.sum(diff * diff * m)
    cnt_part = jnp.sum(m)
    bins_part = lax.dot_general(onehot, m.reshape(BLK, 1), dn,
                                preferred_element_type=f32)

    @pl.when(i == 0)
    def _init():
        bins_acc[...] = bins_part
        cnt_acc[0] = cnt_part
        en_acc[0] = en_part

    @pl.when(i > 0)
    def _accum():
        bins_acc[...] += bins_part
        cnt_acc[0] += cnt_part
        en_acc[0] += en_part

    f0 = f0_ref[...]
    f1 = f1_ref[...]
    f2 = f2_ref[...]
    d0 = d0_ref[...]
    d1 = d1_ref[...]
    d2v = d2_ref[...]
    xT_s[0:EDIM, :] = qT
    trip3 = jnp.concatenate([f0, f1, f2], axis=0)
    dir3 = jnp.concatenate([d0, d1, d2v], axis=0)
    row = EDIM
    for trip in (trip3, dir3):
        for p in range(4):
            fq = float(2.0 ** p)
            xT_s[row:row + 3, :] = jnp.sin(fq * trip)
            xT_s[row + 3:row + 6, :] = jnp.cos(fq * trip)
            row += 6
    xT = xT_s[...]

    h = jnp.maximum(lax.dot_general(W1_ref[...], xT, dc,
                                    preferred_element_type=f32)
                    + b1_ref[...], 0.0)
    h = jnp.maximum(lax.dot_general(W2_ref[...], h, dc,
                                    preferred_element_type=f32)
                    + b2_ref[...], 0.0)
    sh = lax.dot_general(W3_ref[...], h, dc,
                         preferred_element_type=f32) + b3_ref[...]

    sig_ref[...] = jnp.maximum(sh[FDIM - 1:FDIM, :], 0.0) * m

    C0 = 0.28209479177387814
    C1 = 0.4886025119029199
    C2 = (1.0925484305920792, -1.0925484305920792, 0.31539156525252005,
          -1.0925484305920792, 0.5462742152960396)
    x, y, z = d0, d1, d2v
    shm = jnp.concatenate(
        [C0 * jnp.ones((1, BLK), f32), -C1 * y, C1 * z, -C1 * x,
         C2[0] * x * y, C2[1] * y * z,
         C2[2] * (2.0 * z * z - x * x - y * y),
         C2[3] * x * z, C2[4] * (x * x - y * y)], axis=0)
    for ci, rref in enumerate((r0_ref, r1_ref, r2_ref)):
        acc = jnp.sum(shm * sh[ci * 9:ci * 9 + 9, :], axis=0, keepdims=True)
        rref[...] = jnp.clip(acc * m + 0.5, 0.0, 1.0)

    @pl.when(i == NBLK_B - 1)
    def _final():
        cnt = cnt_acc[0]
        comm_ref[...] = jnp.reshape(CCOST * en_acc[0] / (cnt * EDIM), (1, 1))
        p = bins_acc[...] / cnt
        perp_ref[...] = jnp.reshape(
            jnp.exp(-jnp.sum(p * jnp.log(p + 1e-10))), (1, 1))


def _stage_b(cT, m, f0, f1, f2, d0, d1, d2, emb, W1, b1, W2, b2, W3, b3):
    f32 = jnp.float32
    flat = pl.BlockSpec((1, BLK), lambda i: (0, i))

    def full(shape):
        return pl.BlockSpec(shape, lambda i, _s=shape: tuple(0 for _ in _s))

    outs = ([jax.ShapeDtypeStruct((1, NT), f32)] * 4
            + [jax.ShapeDtypeStruct((1, 1), f32)] * 2)
    return pl.pallas_call(
        _b_body,
        grid=(NBLK_B,),
        in_specs=[pl.BlockSpec((EDIM, BLK), lambda i: (0, i)),
                  flat, flat, flat, flat, flat, flat, flat,
                  full((NEMB, EDIM)), full((112, HID)), full((HID, 1)),
                  full((HID, HID)), full((HID, 1)),
                  full((HID, FDIM)), full((FDIM, 1))],
        out_specs=[flat] * 4 + [pl.BlockSpec((1, 1), lambda i: (0, 0))] * 2,
        out_shape=outs,
        scratch_shapes=[pltpu.VMEM((NEMB, 1), f32),
                        pltpu.SMEM((1,), f32),
                        pltpu.SMEM((1,), f32),
                        pltpu.VMEM((112, BLK), f32)],
    )(cT, m, f0, f1, f2, d0, d1, d2, emb, W1, b1, W2, b2, W3, b3)


def _c_body(sig_ref, r0_ref, r1_ref, r2_ref, t_ref, comp_ref, depth_ref):
    sigma = sig_ref[...]
    alpha = 1.0 - jnp.exp(-sigma * STEP)
    f = 1.0 - alpha + 1e-10
    lf = jnp.log(f)
    cs = lf
    sh_amt = 1
    while sh_amt < NP:
        shifted = jnp.concatenate(
            [jnp.zeros((RA, sh_amt), jnp.float32), cs[:, :NP - sh_amt]],
            axis=1)
        cs = cs + shifted
        sh_amt *= 2
    trans = jnp.exp(cs - lf)
    w = alpha * trans
    acc = jnp.sum(w, axis=1, keepdims=True)
    bg = 1.0 - acc
    comps = [jnp.sum(w * r_ref[...], axis=1, keepdims=True) + bg
             for r_ref in (r0_ref, r1_ref, r2_ref)]
    comp_ref[...] = jnp.concatenate(comps, axis=1)
    depth_ref[...] = jnp.sum(w * t_ref[...], axis=1, keepdims=True)


def _stage_c(sigma, r0, r1, r2, tvals):
    f32 = jnp.float32
    spec = pl.BlockSpec((RA, NP), lambda i: (i, 0))
    return pl.pallas_call(
        _c_body,
        grid=(NRAYS // RA,),
        in_specs=[spec] * 5,
        out_specs=[pl.BlockSpec((RA, 3), lambda i: (i, 0)),
                   pl.BlockSpec((RA, 1), lambda i: (i, 0))],
        out_shape=[jax.ShapeDtypeStruct((NRAYS, 3), f32),
                   jax.ShapeDtypeStruct((NRAYS, 1), f32)],
    )(sigma, r0, r1, r2, tvals)


def kernel(rays_o, rays_d, grid_id, data, emb, W1, b1, W2, b2, W3, b3):
    del grid_id
    (base, wx, wy, wz, maskf, f0, f1, f2, d0, d1, d2, tvals) = \
        _stage_a(rays_o, rays_d)
    tbl = data.reshape(4, 16 * NCELL)
    cT = _sc_gather(tbl, base.reshape(NT), wx.reshape(NT),
                    wy.reshape(NT), wz.reshape(NT))
    sigma, r0, r1, r2, comm, perp = _stage_b(
        cT, maskf.reshape(1, NT), f0.reshape(1, NT), f1.reshape(1, NT),
        f2.reshape(1, NT), d0.reshape(1, NT), d1.reshape(1, NT),
        d2.reshape(1, NT),
        emb, W1, b1.reshape(HID, 1), W2, b2.reshape(HID, 1),
        W3, b3.reshape(FDIM, 1))
    comp, depth = _stage_c(sigma.reshape(NRAYS, NP), r0.reshape(NRAYS, NP),
                           r1.reshape(NRAYS, NP), r2.reshape(NRAYS, NP),
                           tvals)
    return comp, depth.reshape(NRAYS), comm[0, 0], perp[0, 0]


# speedup vs baseline: 1.0946x; 1.0946x over previous
import math

import jax
import jax.numpy as jnp
from jax import lax
from jax.experimental import pallas as pl
from jax.experimental.pallas import tpu as pltpu
from jax.experimental.pallas import tpu_sc as plsc

RESO = 16
EDIM = 64
NEMB = 512
CCOST = 0.25
RAD = 1.3
SH_DEG = 2
FDIM = (SH_DEG + 1) ** 2 * 3 + 1
HID = 32
NRAYS = 1024

_units = RAD * 2 / (RESO - 1)
STEP = _units / 8.0
N_INTRS = int(math.sqrt(3) * RAD * 2 / STEP) - 1
NP = 208
NT = NRAYS * NP
NCELL = RESO * RESO * RESO

RA = 64
BLK = 1664
NBLK_B = NT // BLK
NW = 32
PT_W = NT // NW
SCH = 1664
NSCH = PT_W // SCH
NGRP = SCH // 16

_CORNERS = ((0, 0, 0), (1, 0, 0), (0, 1, 0), (1, 1, 0),
            (0, 0, 1), (1, 0, 1), (0, 1, 1), (1, 1, 1))


def _a_body(ro_ref, rd_ref, base_ref, wx_ref, wy_ref, wz_ref, m_ref,
            f0_ref, f1_ref, f2_ref, d0_ref, d1_ref, d2_ref, t_ref):
    ro = ro_ref[...]
    rd = rd_ref[...]
    ox, oy, oz = ro[:, 0:1], ro[:, 1:2], ro[:, 2:3]
    rx, ry, rz = rd[:, 0:1], rd[:, 1:2], rd[:, 2:3]
    norm = jnp.sqrt(rx * rx + ry * ry + rz * rz)
    dnx, dny, dnz = rx / norm, ry / norm, rz / norm

    def axis_ts(o, d):
        sd = jnp.where(jnp.abs(d) < 1e-9, 1e-9, d)
        inv = 1.0 / sd
        t1 = (-RAD - o) * inv
        t2 = (RAD - o) * inv
        return jnp.minimum(t1, t2), jnp.maximum(t1, t2)

    mn0, mx0 = axis_ts(ox, dnx)
    mn1, mx1 = axis_ts(oy, dny)
    mn2, mx2 = axis_ts(oz, dnz)
    tmin = jnp.maximum(jnp.maximum(jnp.maximum(mn0, mn1), mn2), 0.0)
    tmax = jnp.minimum(jnp.minimum(mx0, mx1), mx2)

    lane = lax.broadcasted_iota(jnp.int32, (RA, NP), 1)
    offs = lane.astype(jnp.float32) * STEP
    t = tmin + offs
    px = ox + dnx * t
    py = oy + dny * t
    pz = oz + dnz * t
    mask = ((t < tmax) & (tmax > tmin)
            & (jnp.abs(px) <= RAD) & (jnp.abs(py) <= RAD)
            & (jnp.abs(pz) <= RAD) & (lane < N_INTRS))
    zero = jnp.zeros((RA, NP), jnp.float32)
    px = jnp.where(mask, px, zero)
    py = jnp.where(mask, py, zero)
    pz = jnp.where(mask, pz, zero)

    def cell(p):
        coords = (p / RAD + 1.0) * (0.5 * (RESO - 1))
        c0 = jnp.clip(jnp.floor(coords), 0.0, RESO - 2)
        return c0.astype(jnp.int32), coords - c0

    cx, wx = cell(px)
    cy, wy = cell(py)
    cz, wz = cell(pz)
    base_ref[...] = cz * (RESO * RESO) + cy * RESO + cx
    wx_ref[...] = wx
    wy_ref[...] = wy
    wz_ref[...] = wz
    m_ref[...] = mask.astype(jnp.float32)

    def fine(p):
        pcc = p * (RESO / (RAD * 2)) + RESO / 2
        return (pcc - jnp.floor(pcc)) * 2.0 - 1.0

    f0_ref[...] = fine(px)
    f1_ref[...] = fine(py)
    f2_ref[...] = fine(pz)
    d0_ref[...] = jnp.broadcast_to(dnx, (RA, NP))
    d1_ref[...] = jnp.broadcast_to(dny, (RA, NP))
    d2_ref[...] = jnp.broadcast_to(dnz, (RA, NP))
    t_ref[...] = t


def _stage_a(rays_o, rays_d):
    n_grid = NRAYS // RA
    f32 = jnp.float32
    outs = [jax.ShapeDtypeStruct((NRAYS, NP), jnp.int32)] + \
           [jax.ShapeDtypeStruct((NRAYS, NP), f32)] * 11
    spec = pl.BlockSpec((RA, NP), lambda i: (i, 0))
    return pl.pallas_call(
        _a_body,
        grid=(n_grid,),
        in_specs=[pl.BlockSpec((RA, 3), lambda i: (i, 0))] * 2,
        out_specs=[spec] * 12,
        out_shape=outs,
    )(rays_o, rays_d)


def _sc_body(tbl_hbm, idx_hbm, wx_hbm, wy_hbm, wz_hbm, out_hbm,
             tblq, idxv, wxv, wyv, wzv, outv):
    wid = lax.axis_index("s") * 2 + lax.axis_index("c")
    tile_base = wid * PT_W

    for q in range(4):
        pltpu.sync_copy(tbl_hbm.at[q], tblq)
        for s in range(NSCH):
            base = tile_base + s * SCH
            pltpu.sync_copy(idx_hbm.at[pl.ds(base, SCH)], idxv)
            pltpu.sync_copy(wx_hbm.at[pl.ds(base, SCH)], wxv)
            pltpu.sync_copy(wy_hbm.at[pl.ds(base, SCH)], wyv)
            pltpu.sync_copy(wz_hbm.at[pl.ds(base, SCH)], wzv)

            def body(g, carry):
                off = g * 16
                cellv = idxv[pl.ds(off, 16)]
                wx = wxv[pl.ds(off, 16)]
                wy = wyv[pl.ds(off, 16)]
                wz = wzv[pl.ds(off, 16)]
                ux = 1.0 - wx
                uy = 1.0 - wy
                uz = 1.0 - wz
                pxy = (ux * uy, wx * uy, ux * wy, wx * wy)
                wcorn = [pxy[0] * uz, pxy[1] * uz, pxy[2] * uz, pxy[3] * uz,
                         pxy[0] * wz, pxy[1] * wz, pxy[2] * wz, pxy[3] * wz]
                accs = [None] * 16
                for k, (dx, dy, dz) in enumerate(_CORNERS):
                    cidx = cellv + (dz * (RESO * RESO) + dy * RESO + dx)
                    wk = wcorn[k]
                    for c in range(16):
                        val = plsc.load_gather(tblq, [cidx + c * NCELL])
                        accs[c] = val * wk if k == 0 else accs[c] + val * wk
                for c in range(16):
                    outv[c, pl.ds(off, 16)] = accs[c]
                return carry

            lax.fori_loop(0, NGRP, body, 0)
            pltpu.sync_copy(outv,
                            out_hbm.at[pl.ds(q * 16, 16), pl.ds(base, SCH)])


def _sc_gather(tbl, idx, wx, wy, wz):
    f32 = jnp.float32
    mesh = plsc.VectorSubcoreMesh(core_axis_name="c", subcore_axis_name="s",
                                  num_cores=2, num_subcores=16)
    fn = pl.kernel(
        _sc_body,
        out_type=jax.ShapeDtypeStruct((EDIM, NT), f32),
        mesh=mesh,
        compiler_params=pltpu.CompilerParams(needs_layout_passes=False),
        scratch_types=[
            pltpu.VMEM((16 * NCELL,), f32),
            pltpu.VMEM((SCH,), jnp.int32),
            pltpu.VMEM((SCH,), f32),
            pltpu.VMEM((SCH,), f32),
            pltpu.VMEM((SCH,), f32),
            pltpu.VMEM((16, SCH), f32),
        ],
    )
    return fn(tbl, idx, wx, wy, wz)


def _b_body(cT_ref, m_ref, fc3_ref, dir3_ref,
            emb_ref, W1q_ref, W1ps_ref, W1pc_ref, W1ds_ref, W1dc_ref,
            b1_ref, W2_ref, b2_ref, W3_ref, b3_ref,
            sig_ref, r0_ref, r1_ref, r2_ref, comm_ref, perp_ref,
            bins_acc, cnt_acc, en_acc):
    i = pl.program_id(0)
    f32 = jnp.float32
    cT = cT_ref[...]
    m = m_ref[...]
    emb = emb_ref[...]

    dn = (((1,), (0,)), ((), ()))
    scores = lax.dot_general(emb.astype(jnp.bfloat16), cT.astype(jnp.bfloat16),
                             dn, preferred_element_type=f32)
    e2 = jnp.sum(emb * emb, axis=1)[:, None]
    d2 = e2 - 2.0 * scores
    minv = jnp.min(d2, axis=0, keepdims=True)
    iota = lax.broadcasted_iota(jnp.int32, (NEMB, BLK), 0)
    idx = jnp.min(jnp.where(d2 == minv, iota, NEMB), axis=0, keepdims=True)
    onehot = (iota == idx).astype(f32)

    dc = (((0,), (0,)), ((), ()))
    qT = lax.dot_general(emb, onehot, dc, preferred_element_type=f32)

    diff = qT - cT
    en_part = jnp.sum(diff * diff * m)
    cnt_part = jnp.sum(m)
    bins_part = lax.dot_general(onehot, m.reshape(BLK, 1), dn,
                                preferred_element_type=f32)

    @pl.when(i == 0)
    def _init():
        bins_acc[...] = bins_part
        cnt_acc[0] = cnt_part
        en_acc[0] = en_part

    @pl.when(i > 0)
    def _accum():
        bins_acc[...] += bins_part
        cnt_acc[0] += cnt_part
        en_acc[0] += en_part

    trip3 = fc3_ref[...]
    dir3 = dir3_ref[...]
    t12 = jnp.concatenate([trip3, 2.0 * trip3, 4.0 * trip3, 8.0 * trip3],
                          axis=0)
    d12 = jnp.concatenate([dir3, 2.0 * dir3, 4.0 * dir3, 8.0 * dir3],
                          axis=0)
    h = (lax.dot_general(W1q_ref[...], qT, dc, preferred_element_type=f32)
         + lax.dot_general(W1ps_ref[...], jnp.sin(t12), dc,
                           preferred_element_type=f32)
         + lax.dot_general(W1pc_ref[...], jnp.cos(t12), dc,
                           preferred_element_type=f32)
         + lax.dot_general(W1ds_ref[...], jnp.sin(d12), dc,
                           preferred_element_type=f32)
         + lax.dot_general(W1dc_ref[...], jnp.cos(d12), dc,
                           preferred_element_type=f32))
    h = jnp.maximum(h + b1_ref[...], 0.0)
    h = jnp.maximum(lax.dot_general(W2_ref[...], h, dc,
                                    preferred_element_type=f32)
                    + b2_ref[...], 0.0)
    sh = lax.dot_general(W3_ref[...], h, dc,
                         preferred_element_type=f32) + b3_ref[...]

    sig_ref[...] = jnp.maximum(sh[FDIM - 1:FDIM, :], 0.0) * m

    C0 = 0.28209479177387814
    C1 = 0.4886025119029199
    C2 = (1.0925484305920792, -1.0925484305920792, 0.31539156525252005,
          -1.0925484305920792, 0.5462742152960396)
    x = dir3[0:1, :]
    y = dir3[1:2, :]
    z = dir3[2:3, :]
    shm = jnp.concatenate(
        [C0 * jnp.ones((1, BLK), f32), -C1 * y, C1 * z, -C1 * x,
         C2[0] * x * y, C2[1] * y * z,
         C2[2] * (2.0 * z * z - x * x - y * y),
         C2[3] * x * z, C2[4] * (x * x - y * y)], axis=0)
    for ci, rref in enumerate((r0_ref, r1_ref, r2_ref)):
        acc = jnp.sum(shm * sh[ci * 9:ci * 9 + 9, :], axis=0, keepdims=True)
        rref[...] = jnp.clip(acc * m + 0.5, 0.0, 1.0)

    @pl.when(i == NBLK_B - 1)
    def _final():
        cnt = cnt_acc[0]
        comm_ref[...] = jnp.reshape(CCOST * en_acc[0] / (cnt * EDIM), (1, 1))
        p = bins_acc[...] / cnt
        perp_ref[...] = jnp.reshape(
            jnp.exp(-jnp.sum(p * jnp.log(p + 1e-10))), (1, 1))


def _stage_b(cT, m, fc3, dir3, emb, W1q, W1ps, W1pc, W1ds, W1dc,
             b1, W2, b2, W3, b3):
    f32 = jnp.float32
    flat = pl.BlockSpec((1, BLK), lambda i: (0, i))
    slab = pl.BlockSpec((3, BLK), lambda i: (0, i))

    def full(shape):
        return pl.BlockSpec(shape, lambda i, _s=shape: tuple(0 for _ in _s))

    outs = ([jax.ShapeDtypeStruct((1, NT), f32)] * 4
            + [jax.ShapeDtypeStruct((1, 1), f32)] * 2)
    return pl.pallas_call(
        _b_body,
        grid=(NBLK_B,),
        in_specs=[pl.BlockSpec((EDIM, BLK), lambda i: (0, i)),
                  flat, slab, slab,
                  full((NEMB, EDIM)), full((EDIM, HID)),
                  full((12, HID)), full((12, HID)),
                  full((12, HID)), full((12, HID)), full((HID, 1)),
                  full((HID, HID)), full((HID, 1)),
                  full((HID, FDIM)), full((FDIM, 1))],
        out_specs=[flat] * 4 + [pl.BlockSpec((1, 1), lambda i: (0, 0))] * 2,
        out_shape=outs,
        scratch_shapes=[pltpu.VMEM((NEMB, 1), f32),
                        pltpu.SMEM((1,), f32),
                        pltpu.SMEM((1,), f32)],
    )(cT, m, fc3, dir3, emb, W1q, W1ps, W1pc, W1ds, W1dc,
      b1, W2, b2, W3, b3)


def _c_body(sig_ref, r0_ref, r1_ref, r2_ref, t_ref, comp_ref, depth_ref):
    sigma = sig_ref[...]
    alpha = 1.0 - jnp.exp(-sigma * STEP)
    f = 1.0 - alpha + 1e-10
    lf = jnp.log(f)
    cs = lf
    sh_amt = 1
    while sh_amt < NP:
        shifted = jnp.concatenate(
            [jnp.zeros((RA, sh_amt), jnp.float32), cs[:, :NP - sh_amt]],
            axis=1)
        cs = cs + shifted
        sh_amt *= 2
    trans = jnp.exp(cs - lf)
    w = alpha * trans
    acc = jnp.sum(w, axis=1, keepdims=True)
    bg = 1.0 - acc
    comps = [jnp.sum(w * r_ref[...], axis=1, keepdims=True) + bg
             for r_ref in (r0_ref, r1_ref, r2_ref)]
    comp_ref[...] = jnp.concatenate(comps, axis=1)
    depth_ref[...] = jnp.sum(w * t_ref[...], axis=1, keepdims=True)


def _stage_c(sigma, r0, r1, r2, tvals):
    f32 = jnp.float32
    spec = pl.BlockSpec((RA, NP), lambda i: (i, 0))
    return pl.pallas_call(
        _c_body,
        grid=(NRAYS // RA,),
        in_specs=[spec] * 5,
        out_specs=[pl.BlockSpec((RA, 3), lambda i: (i, 0)),
                   pl.BlockSpec((RA, 1), lambda i: (i, 0))],
        out_shape=[jax.ShapeDtypeStruct((NRAYS, 3), f32),
                   jax.ShapeDtypeStruct((NRAYS, 1), f32)],
    )(sigma, r0, r1, r2, tvals)


def kernel(rays_o, rays_d, grid_id, data, emb, W1, b1, W2, b2, W3, b3):
    del grid_id
    (base, wx, wy, wz, maskf, f0, f1, f2, d0, d1, d2, tvals) = \
        _stage_a(rays_o, rays_d)
    tbl = data.reshape(4, 16 * NCELL)
    cT = _sc_gather(tbl, base.reshape(NT), wx.reshape(NT),
                    wy.reshape(NT), wz.reshape(NT))
    fc3 = jnp.stack([f0, f1, f2]).reshape(3, NT)
    dir3 = jnp.stack([d0, d1, d2]).reshape(3, NT)
    perm_sin = jnp.array([6 * p + j for p in range(4) for j in range(3)])
    perm_cos = perm_sin + 3
    W1p = W1[EDIM:EDIM + 24]
    W1d = W1[EDIM + 24:EDIM + 48]
    sigma, r0, r1, r2, comm, perp = _stage_b(
        cT, maskf.reshape(1, NT), fc3, dir3,
        emb, W1[0:EDIM], W1p[perm_sin], W1p[perm_cos],
        W1d[perm_sin], W1d[perm_cos],
        b1.reshape(HID, 1), W2, b2.reshape(HID, 1),
        W3, b3.reshape(FDIM, 1))
    comp, depth = _stage_c(sigma.reshape(NRAYS, NP), r0.reshape(NRAYS, NP),
                           r1.reshape(NRAYS, NP), r2.reshape(NRAYS, NP),
                           tvals)
    return comp, depth.reshape(NRAYS), comm[0, 0], perp[0, 0]
